# baseline (speedup 1.0000x reference)
import functools

import jax
import jax.numpy as jnp
from jax import lax
from jax.experimental import pallas as pl
from jax.experimental.pallas import tpu as pltpu
from jax.experimental.pallas import tpu_sc as plsc

N_NODES = 10000
N_EDGES = 320000
HID = 128

NC = 2
NS = 16
NW = NC * NS

N_PAD = 10112
ROWS_PER_TILE = N_PAD // NS

CH = 128
E_TILE = 10240
NCH = E_TILE // CH
NPAIR = NCH // 2
N_EDGES_P = NW * E_TILE
JUNK_ROW = 10016

DEG_W = 128
D_TILE = N_EDGES_P // NS
DCH = 128
NDCH = D_TILE // DCH


@functools.cache
def _mesh():
    return plsc.VectorSubcoreMesh(
        core_axis_name="c", subcore_axis_name="s", num_cores=NC, num_subcores=NS
    )


_WB_CHUNKS = [(0, 128), (128, 128), (256, 128), (384, 128), (512, 120)]


def _deg_body(ids_hbm, ones_hbm, zeros_hbm, out_hbm, idx_v, ones_v, zbuf_v, hist_sh):
    c = lax.axis_index("c")
    s = lax.axis_index("s")
    pltpu.sync_copy(zeros_hbm, zbuf_v)
    row0 = s * ROWS_PER_TILE
    for off, sz in _WB_CHUNKS:
        pltpu.sync_copy(zbuf_v.at[pl.ds(0, sz)], hist_sh.at[pl.ds(row0 + off, sz)])
    pltpu.sync_copy(ones_hbm, ones_v)
    plsc.subcore_barrier()
    my_ids = ids_hbm.at[c]

    def body(i, carry):
        pltpu.sync_copy(my_ids.at[pl.ds(s * D_TILE + i * DCH, DCH)], idx_v)
        pltpu.sync_copy(ones_v, hist_sh.at[idx_v], add=True)
        return carry

    lax.fori_loop(0, NDCH, body, 0)
    plsc.subcore_barrier()
    for off, sz in _WB_CHUNKS:
        pltpu.sync_copy(hist_sh.at[pl.ds(row0 + off, sz)], zbuf_v.at[pl.ds(0, sz)])
        pltpu.sync_copy(zbuf_v.at[pl.ds(0, sz)], out_hbm.at[c, pl.ds(row0 + off, sz)])


def _msgpass_body(h_hbm, sd_hbm, zeros_hbm, out_hbm,
                  sd_v, msg_v, agg_sh, sem_i, sem_g0, sem_g1, sem_s0, sem_s1):
    c = lax.axis_index("c")
    s = lax.axis_index("s")
    wid = c * NS + s
    row0 = s * ROWS_PER_TILE
    pltpu.sync_copy(zeros_hbm, msg_v.at[0])
    for off, sz in _WB_CHUNKS:
        pltpu.sync_copy(msg_v.at[0, pl.ds(0, sz)], agg_sh.at[pl.ds(row0 + off, sz)])
    plsc.subcore_barrier()

    def gstart(q, j, b, sem):
        pltpu.async_copy(h_hbm.at[sd_v.at[q, j, 0]], msg_v.at[b], sem)

    def gwait(sem):
        pltpu.make_async_copy(h_hbm.at[sd_v.at[0, 0, 0]], msg_v.at[0], sem).wait()

    def sstart(q, j, b, sem):
        pltpu.async_copy(msg_v.at[b], agg_sh.at[sd_v.at[q, j, 1]], sem, add=True)

    def swait(sem):
        pltpu.make_async_copy(msg_v.at[0], agg_sh.at[sd_v.at[0, 0, 1]], sem).wait()

    def istart(p, slot):
        pltpu.async_copy(sd_hbm.at[wid, p], sd_v.at[slot], sem_i)

    def iwait():
        pltpu.make_async_copy(sd_hbm.at[0, 0], sd_v.at[0], sem_i).wait()

    pltpu.sync_copy(sd_hbm.at[wid, 0], sd_v.at[0])
    gstart(0, 0, 0, sem_g0)
    istart(1, 1)
    gstart(0, 1, 1, sem_g1)
    gwait(sem_g0)
    sstart(0, 0, 0, sem_s0)
    iwait()
    swait(sem_s0)
    gstart(1, 0, 0, sem_g0)
    gwait(sem_g1)
    sstart(0, 1, 1, sem_s1)

    def body(p, carry):
        q = p % 2
        swait(sem_s1)
        istart(p + 1, 1 - q)
        gstart(q, 1, 1, sem_g1)
        gwait(sem_g0)
        sstart(q, 0, 0, sem_s0)
        iwait()
        swait(sem_s0)
        gstart(1 - q, 0, 0, sem_g0)
        gwait(sem_g1)
        sstart(q, 1, 1, sem_s1)
        return carry

    lax.fori_loop(1, NPAIR - 1, body, 0)

    swait(sem_s1)
    gstart(1, 1, 1, sem_g1)
    gwait(sem_g0)
    sstart(1, 0, 0, sem_s0)
    gwait(sem_g1)
    sstart(1, 1, 1, sem_s1)
    swait(sem_s0)
    swait(sem_s1)

    plsc.subcore_barrier()
    for off, sz in _WB_CHUNKS:
        pltpu.sync_copy(agg_sh.at[pl.ds(row0 + off, sz)], msg_v.at[0, pl.ds(0, sz)])
        pltpu.sync_copy(msg_v.at[0, pl.ds(0, sz)], out_hbm.at[c, pl.ds(row0 + off, sz)])


@functools.cache
def _deg_kernel():
    return pl.kernel(
        _deg_body,
        out_type=jax.ShapeDtypeStruct((NC, N_PAD, DEG_W), jnp.float32),
        mesh=_mesh(),
        scratch_types=[
            pltpu.VMEM((DCH,), jnp.int32),
            pltpu.VMEM((DCH, DEG_W), jnp.float32),
            pltpu.VMEM((DCH, DEG_W), jnp.float32),
            pltpu.VMEM_SHARED((N_PAD, DEG_W), jnp.float32),
        ],
    )


@functools.cache
def _msgpass_kernel():
    return pl.kernel(
        _msgpass_body,
        out_type=jax.ShapeDtypeStruct((NC, N_PAD, HID), jnp.float32),
        mesh=_mesh(),
        scratch_types=[
            pltpu.VMEM((2, 2, 2, CH), jnp.int32),
            pltpu.VMEM((2, CH, HID), jnp.float32),
            pltpu.VMEM_SHARED((N_PAD, HID), jnp.float32),
            pltpu.SemaphoreType.DMA,
            pltpu.SemaphoreType.DMA,
            pltpu.SemaphoreType.DMA,
            pltpu.SemaphoreType.DMA,
            pltpu.SemaphoreType.DMA,
        ],
    )


def _proj_body(f0_ref, f1_ref, f2_ref, w0_ref, b0_ref, w1_ref, b1_ref,
               w2_ref, b2_ref, hist_ref, hh_ref, ns_ref, nd_ref):
    h0 = jnp.dot(f0_ref[...], w0_ref[...], preferred_element_type=jnp.float32) + b0_ref[...]
    h1 = jnp.dot(f1_ref[...], w1_ref[...], preferred_element_type=jnp.float32) + b1_ref[...]
    h2 = jnp.dot(f2_ref[...], w2_ref[...], preferred_element_type=jnp.float32) + b2_ref[...]
    h = jnp.concatenate([h0, h1, h2], axis=0)
    out_deg = hist_ref[0, :N_NODES, 0]
    in_deg = hist_ref[1, :N_NODES, 0]
    ns = lax.rsqrt(jnp.maximum(out_deg, 1.0))
    nd = lax.rsqrt(jnp.maximum(in_deg, 1.0))
    hh_ref[...] = h * ns[:, None]
    ns_ref[...] = ns[:, None]
    nd_ref[...] = nd[:, None]


def _scale_body(p_ref, nd_ref, ns_ref, out_ref):
    a = p_ref[0, :N_NODES, :] + p_ref[1, :N_NODES, :]
    out_ref[...] = jnp.maximum(a * nd_ref[...], 0.0) * ns_ref[...]


def _final_body(q_ref, nd_ref, w2_ref, b2_ref, out_ref):
    a = q_ref[0, :N_NODES, :] + q_ref[1, :N_NODES, :]
    a = a * nd_ref[...]
    z = jnp.dot(a, w2_ref[...], preferred_element_type=jnp.float32) + b2_ref[...]
    out_ref[...] = jnp.maximum(z, 0.0)


def kernel(feat0, feat1, feat2, edge_index, Wf0, bf0, Wf1, bf1, Wf2, bf2,
           W2, b2, semantic_weight):
    src = edge_index[0].astype(jnp.int32)
    dst = edge_index[1].astype(jnp.int32)
    n_pad_edges = N_EDGES_P - N_EDGES
    junk = jnp.full((n_pad_edges,), JUNK_ROW, jnp.int32)
    src_mp = jnp.concatenate([src, jnp.zeros((n_pad_edges,), jnp.int32)])
    dst_mp = jnp.concatenate([dst, junk])
    srcr = src_mp.reshape(NW, NPAIR, 2, CH)
    dstr = dst_mp.reshape(NW, NPAIR, 2, CH)
    sd = jnp.stack([srcr, dstr], axis=3)
    ids = jnp.stack([jnp.concatenate([src, junk]), jnp.concatenate([dst, junk])])

    ones_deg = jnp.ones((DCH, DEG_W), jnp.float32)
    zeros_deg = jnp.zeros((DCH, DEG_W), jnp.float32)
    zeros_mp = jnp.zeros((CH, HID), jnp.float32)

    hist = _deg_kernel()(ids, ones_deg, zeros_deg)

    hh0, ns, nd = pl.pallas_call(
        _proj_body,
        out_shape=[
            jax.ShapeDtypeStruct((N_NODES, HID), jnp.float32),
            jax.ShapeDtypeStruct((N_NODES, 1), jnp.float32),
            jax.ShapeDtypeStruct((N_NODES, 1), jnp.float32),
        ],
    )(feat0, feat1, feat2, Wf0, bf0, Wf1, bf1, Wf2, bf2, hist)

    p0 = _msgpass_kernel()(hh0, sd, zeros_mp)

    hh1 = pl.pallas_call(
        _scale_body,
        out_shape=jax.ShapeDtypeStruct((N_NODES, HID), jnp.float32),
    )(p0, nd, ns)

    p1 = _msgpass_kernel()(hh1, sd, zeros_mp)

    out = pl.pallas_call(
        _final_body,
        out_shape=jax.ShapeDtypeStruct((N_NODES, HID), jnp.float32),
    )(p1, nd, W2, b2)

    return (out, semantic_weight)


# speedup vs baseline: 2.5780x; 2.5780x over previous
import functools

import jax
import jax.numpy as jnp
from jax import lax
from jax.experimental import pallas as pl
from jax.experimental.pallas import tpu as pltpu
from jax.experimental.pallas import tpu_sc as plsc

N_NODES = 10000
N_EDGES = 320000
HID = 128

NC = 2
NS = 16
NW = NC * NS

N_PAD = 10112
ROWS_PER_TILE = N_PAD // NS

CH = 128
E_TILE = 10240
NCH = E_TILE // CH
NPAIR = NCH // 2
N_EDGES_P = NW * E_TILE

DEG_W = 128
D_TILE = N_EDGES_P // NS
DCH = 128
NDCH = D_TILE // DCH


@functools.cache
def _mesh():
    return plsc.VectorSubcoreMesh(
        core_axis_name="c", subcore_axis_name="s", num_cores=NC, num_subcores=NS
    )


_WB_CHUNKS = [(0, 128), (128, 128), (256, 128), (384, 128), (512, 120)]


def _deg_body(ids_hbm, ones_hbm, zeros_hbm, out_hbm, idx_v, ones_v, zbuf_v, hist_sh):
    c = lax.axis_index("c")
    s = lax.axis_index("s")
    pltpu.sync_copy(zeros_hbm, zbuf_v)
    row0 = s * ROWS_PER_TILE
    for off, sz in _WB_CHUNKS:
        pltpu.sync_copy(zbuf_v.at[pl.ds(0, sz)], hist_sh.at[pl.ds(row0 + off, sz)])
    pltpu.sync_copy(ones_hbm, ones_v)
    plsc.subcore_barrier()
    my_ids = ids_hbm.at[c]

    def body(i, carry):
        pltpu.sync_copy(my_ids.at[pl.ds(s * D_TILE + i * DCH, DCH)], idx_v)
        pltpu.sync_copy(ones_v, hist_sh.at[idx_v], add=True)
        return carry

    lax.fori_loop(0, NDCH, body, 0)
    plsc.subcore_barrier()
    for off, sz in _WB_CHUNKS:
        pltpu.sync_copy(hist_sh.at[pl.ds(row0 + off, sz)], zbuf_v.at[pl.ds(0, sz)])
        pltpu.sync_copy(zbuf_v.at[pl.ds(0, sz)], out_hbm.at[c, pl.ds(row0 + off, sz)])


def _msgpass_body(h_hbm, sd_hbm, zeros_hbm, out_hbm,
                  sd_v, msg_v, agg_sh, sem_i, sem_g0, sem_g1, sem_s0, sem_s1):
    c = lax.axis_index("c")
    s = lax.axis_index("s")
    wid = c * NS + s
    row0 = s * ROWS_PER_TILE
    pltpu.sync_copy(zeros_hbm, msg_v.at[0])
    for off, sz in _WB_CHUNKS:
        pltpu.sync_copy(msg_v.at[0, pl.ds(0, sz)], agg_sh.at[pl.ds(row0 + off, sz)])
    plsc.subcore_barrier()

    def gstart(q, j, b, sem):
        pltpu.async_copy(h_hbm.at[sd_v.at[q, j, 0]], msg_v.at[b], sem)

    def gwait(sem):
        pltpu.make_async_copy(h_hbm.at[sd_v.at[0, 0, 0]], msg_v.at[0], sem).wait()

    def sstart(q, j, b, sem):
        pltpu.async_copy(msg_v.at[b], agg_sh.at[sd_v.at[q, j, 1]], sem, add=True)

    def swait(sem):
        pltpu.make_async_copy(msg_v.at[0], agg_sh.at[sd_v.at[0, 0, 1]], sem).wait()

    def istart(p, slot):
        pltpu.async_copy(sd_hbm.at[wid, p], sd_v.at[slot], sem_i)

    def iwait():
        pltpu.make_async_copy(sd_hbm.at[0, 0], sd_v.at[0], sem_i).wait()

    pltpu.sync_copy(sd_hbm.at[wid, 0], sd_v.at[0])
    gstart(0, 0, 0, sem_g0)
    istart(1, 1)
    gstart(0, 1, 1, sem_g1)
    gwait(sem_g0)
    sstart(0, 0, 0, sem_s0)
    iwait()
    swait(sem_s0)
    gstart(1, 0, 0, sem_g0)
    gwait(sem_g1)
    sstart(0, 1, 1, sem_s1)

    def body(p, carry):
        q = p % 2
        swait(sem_s1)
        istart(p + 1, 1 - q)
        gstart(q, 1, 1, sem_g1)
        gwait(sem_g0)
        sstart(q, 0, 0, sem_s0)
        iwait()
        swait(sem_s0)
        gstart(1 - q, 0, 0, sem_g0)
        gwait(sem_g1)
        sstart(q, 1, 1, sem_s1)
        return carry

    lax.fori_loop(1, NPAIR - 1, body, 0)

    swait(sem_s1)
    gstart(1, 1, 1, sem_g1)
    gwait(sem_g0)
    sstart(1, 0, 0, sem_s0)
    gwait(sem_g1)
    sstart(1, 1, 1, sem_s1)
    swait(sem_s0)
    swait(sem_s1)

    plsc.subcore_barrier()
    for off, sz in _WB_CHUNKS:
        pltpu.sync_copy(agg_sh.at[pl.ds(row0 + off, sz)], msg_v.at[0, pl.ds(0, sz)])
        pltpu.sync_copy(msg_v.at[0, pl.ds(0, sz)], out_hbm.at[c, pl.ds(row0 + off, sz)])


@functools.cache
def _deg_kernel():
    return pl.kernel(
        _deg_body,
        out_type=jax.ShapeDtypeStruct((NC, N_PAD, DEG_W), jnp.float32),
        mesh=_mesh(),
        scratch_types=[
            pltpu.VMEM((DCH,), jnp.int32),
            pltpu.VMEM((DCH, DEG_W), jnp.float32),
            pltpu.VMEM((DCH, DEG_W), jnp.float32),
            pltpu.VMEM_SHARED((N_PAD, DEG_W), jnp.float32),
        ],
    )


@functools.cache
def _msgpass_kernel():
    return pl.kernel(
        _msgpass_body,
        out_type=jax.ShapeDtypeStruct((NC, N_PAD, HID), jnp.float32),
        mesh=_mesh(),
        scratch_types=[
            pltpu.VMEM((2, 2, 2, CH), jnp.int32),
            pltpu.VMEM((2, CH, HID), jnp.float32),
            pltpu.VMEM_SHARED((N_PAD, HID), jnp.float32),
            pltpu.SemaphoreType.DMA,
            pltpu.SemaphoreType.DMA,
            pltpu.SemaphoreType.DMA,
            pltpu.SemaphoreType.DMA,
            pltpu.SemaphoreType.DMA,
        ],
    )


def _proj_body(f0_ref, f1_ref, f2_ref, w0_ref, b0_ref, w1_ref, b1_ref,
               w2_ref, b2_ref, hist_ref, hh_ref, ns_ref, nd_ref):
    h0 = jnp.dot(f0_ref[...], w0_ref[...], preferred_element_type=jnp.float32) + b0_ref[...]
    h1 = jnp.dot(f1_ref[...], w1_ref[...], preferred_element_type=jnp.float32) + b1_ref[...]
    h2 = jnp.dot(f2_ref[...], w2_ref[...], preferred_element_type=jnp.float32) + b2_ref[...]
    h = jnp.concatenate([h0, h1, h2], axis=0)
    out_deg = hist_ref[0, :N_NODES, 0]
    in_deg = hist_ref[1, :N_NODES, 0]
    ns = lax.rsqrt(jnp.maximum(out_deg, 1.0))
    nd = lax.rsqrt(jnp.maximum(in_deg, 1.0))
    hh_ref[...] = h * ns[:, None]
    ns_ref[...] = ns[:, None]
    nd_ref[...] = nd[:, None]


def _scale_body(p_ref, nd_ref, ns_ref, out_ref):
    a = p_ref[0, :N_NODES, :] + p_ref[1, :N_NODES, :]
    out_ref[...] = jnp.maximum(a * nd_ref[...], 0.0) * ns_ref[...]


def _final_body(q_ref, nd_ref, w2_ref, b2_ref, out_ref):
    a = q_ref[0, :N_NODES, :] + q_ref[1, :N_NODES, :]
    a = a * nd_ref[...]
    z = jnp.dot(a, w2_ref[...], preferred_element_type=jnp.float32) + b2_ref[...]
    out_ref[...] = jnp.maximum(z, 0.0)


def kernel(feat0, feat1, feat2, edge_index, Wf0, bf0, Wf1, bf1, Wf2, bf2,
           W2, b2, semantic_weight):
    src = edge_index[0].astype(jnp.int32)
    dst = edge_index[1].astype(jnp.int32)
    n_pad_edges = N_EDGES_P - N_EDGES
    pad_iota = jnp.arange(n_pad_edges, dtype=jnp.int32)
    junk = N_NODES + pad_iota % (N_PAD - N_NODES)
    src_mp = jnp.concatenate([src, pad_iota % N_NODES])
    dst_mp = jnp.concatenate([dst, junk])
    srcr = src_mp.reshape(NW, NPAIR, 2, CH)
    dstr = dst_mp.reshape(NW, NPAIR, 2, CH)
    sd = jnp.stack([srcr, dstr], axis=3)
    ids = jnp.stack([jnp.concatenate([src, junk]), jnp.concatenate([dst, junk])])

    ones_deg = jnp.ones((DCH, DEG_W), jnp.float32)
    zeros_deg = jnp.zeros((DCH, DEG_W), jnp.float32)
    zeros_mp = jnp.zeros((CH, HID), jnp.float32)

    hist = _deg_kernel()(ids, ones_deg, zeros_deg)

    hh0, ns, nd = pl.pallas_call(
        _proj_body,
        out_shape=[
            jax.ShapeDtypeStruct((N_NODES, HID), jnp.float32),
            jax.ShapeDtypeStruct((N_NODES, 1), jnp.float32),
            jax.ShapeDtypeStruct((N_NODES, 1), jnp.float32),
        ],
    )(feat0, feat1, feat2, Wf0, bf0, Wf1, bf1, Wf2, bf2, hist)

    p0 = _msgpass_kernel()(hh0, sd, zeros_mp)

    hh1 = pl.pallas_call(
        _scale_body,
        out_shape=jax.ShapeDtypeStruct((N_NODES, HID), jnp.float32),
    )(p0, nd, ns)

    p1 = _msgpass_kernel()(hh1, sd, zeros_mp)

    out = pl.pallas_call(
        _final_body,
        out_shape=jax.ShapeDtypeStruct((N_NODES, HID), jnp.float32),
    )(p1, nd, W2, b2)

    return (out, semantic_weight)


# speedup vs baseline: 3.1223x; 1.2111x over previous
import functools

import jax
import jax.numpy as jnp
from jax import lax
from jax.experimental import pallas as pl
from jax.experimental.pallas import tpu as pltpu
from jax.experimental.pallas import tpu_sc as plsc

N_NODES = 10000
N_EDGES = 320000
HID = 128

NC = 2
NS = 16
NW = NC * NS

N_PAD = 10112
ROWS_PER_TILE = N_PAD // NS

CH = 128
E_TILE = 10240
NCH = E_TILE // CH
NPAIR = NCH // 2
N_EDGES_P = NW * E_TILE

DEG_W = 16
D_TILE = N_EDGES_P // NS
DCH = 128
NDCH = D_TILE // DCH


@functools.cache
def _mesh():
    return plsc.VectorSubcoreMesh(
        core_axis_name="c", subcore_axis_name="s", num_cores=NC, num_subcores=NS
    )


_WB_CHUNKS = [(0, 128), (128, 128), (256, 128), (384, 128), (512, 120)]


def _deg_body(ids_hbm, ones_hbm, zeros_hbm, out_hbm, idx_v, ones_v, zbuf_v, hist_sh):
    c = lax.axis_index("c")
    s = lax.axis_index("s")
    pltpu.sync_copy(zeros_hbm, zbuf_v)
    row0 = s * ROWS_PER_TILE
    for off, sz in _WB_CHUNKS:
        pltpu.sync_copy(zbuf_v.at[pl.ds(0, sz)], hist_sh.at[pl.ds(row0 + off, sz)])
    pltpu.sync_copy(ones_hbm, ones_v)
    plsc.subcore_barrier()
    my_ids = ids_hbm.at[c]

    def body(i, carry):
        pltpu.sync_copy(my_ids.at[pl.ds(s * D_TILE + i * DCH, DCH)], idx_v)
        pltpu.sync_copy(ones_v, hist_sh.at[idx_v], add=True)
        return carry

    lax.fori_loop(0, NDCH, body, 0)
    plsc.subcore_barrier()
    for off, sz in _WB_CHUNKS:
        pltpu.sync_copy(hist_sh.at[pl.ds(row0 + off, sz)], zbuf_v.at[pl.ds(0, sz)])
        pltpu.sync_copy(zbuf_v.at[pl.ds(0, sz)], out_hbm.at[c, pl.ds(row0 + off, sz)])


def _msgpass_body(h_hbm, sd_hbm, zeros_hbm, out_hbm,
                  sd_v, msg_v, agg_sh, sem_i, sem_g0, sem_g1, sem_s0, sem_s1):
    c = lax.axis_index("c")
    s = lax.axis_index("s")
    wid = c * NS + s
    row0 = s * ROWS_PER_TILE
    pltpu.sync_copy(zeros_hbm, msg_v.at[0])
    for off, sz in _WB_CHUNKS:
        pltpu.sync_copy(msg_v.at[0, pl.ds(0, sz)], agg_sh.at[pl.ds(row0 + off, sz)])
    plsc.subcore_barrier()

    def gstart(q, j, b, sem):
        pltpu.async_copy(h_hbm.at[sd_v.at[q, j, 0]], msg_v.at[b], sem)

    def gwait(sem):
        pltpu.make_async_copy(h_hbm.at[sd_v.at[0, 0, 0]], msg_v.at[0], sem).wait()

    def sstart(q, j, b, sem):
        pltpu.async_copy(msg_v.at[b], agg_sh.at[sd_v.at[q, j, 1]], sem, add=True)

    def swait(sem):
        pltpu.make_async_copy(msg_v.at[0], agg_sh.at[sd_v.at[0, 0, 1]], sem).wait()

    def istart(p, slot):
        pltpu.async_copy(sd_hbm.at[wid, p], sd_v.at[slot], sem_i)

    def iwait():
        pltpu.make_async_copy(sd_hbm.at[0, 0], sd_v.at[0], sem_i).wait()

    pltpu.sync_copy(sd_hbm.at[wid, 0], sd_v.at[0])
    gstart(0, 0, 0, sem_g0)
    istart(1, 1)
    gstart(0, 1, 1, sem_g1)
    gwait(sem_g0)
    sstart(0, 0, 0, sem_s0)
    iwait()
    swait(sem_s0)
    gstart(1, 0, 0, sem_g0)
    gwait(sem_g1)
    sstart(0, 1, 1, sem_s1)

    def body(p, carry):
        q = p % 2
        swait(sem_s1)
        istart(p + 1, 1 - q)
        gstart(q, 1, 1, sem_g1)
        gwait(sem_g0)
        sstart(q, 0, 0, sem_s0)
        iwait()
        swait(sem_s0)
        gstart(1 - q, 0, 0, sem_g0)
        gwait(sem_g1)
        sstart(q, 1, 1, sem_s1)
        return carry

    lax.fori_loop(1, NPAIR - 1, body, 0)

    swait(sem_s1)
    gstart(1, 1, 1, sem_g1)
    gwait(sem_g0)
    sstart(1, 0, 0, sem_s0)
    gwait(sem_g1)
    sstart(1, 1, 1, sem_s1)
    swait(sem_s0)
    swait(sem_s1)

    plsc.subcore_barrier()
    for off, sz in _WB_CHUNKS:
        pltpu.sync_copy(agg_sh.at[pl.ds(row0 + off, sz)], msg_v.at[0, pl.ds(0, sz)])
        pltpu.sync_copy(msg_v.at[0, pl.ds(0, sz)], out_hbm.at[c, pl.ds(row0 + off, sz)])


@functools.cache
def _deg_kernel():
    return pl.kernel(
        _deg_body,
        out_type=jax.ShapeDtypeStruct((NC, N_PAD, DEG_W), jnp.float32),
        mesh=_mesh(),
        compiler_params=pltpu.CompilerParams(use_tc_tiling_on_sc=False),
        scratch_types=[
            pltpu.VMEM((DCH,), jnp.int32),
            pltpu.VMEM((DCH, DEG_W), jnp.float32),
            pltpu.VMEM((DCH, DEG_W), jnp.float32),
            pltpu.VMEM_SHARED((N_PAD, DEG_W), jnp.float32),
        ],
    )


@functools.cache
def _msgpass_kernel():
    return pl.kernel(
        _msgpass_body,
        out_type=jax.ShapeDtypeStruct((NC, N_PAD, HID), jnp.float32),
        mesh=_mesh(),
        scratch_types=[
            pltpu.VMEM((2, 2, 2, CH), jnp.int32),
            pltpu.VMEM((2, CH, HID), jnp.float32),
            pltpu.VMEM_SHARED((N_PAD, HID), jnp.float32),
            pltpu.SemaphoreType.DMA,
            pltpu.SemaphoreType.DMA,
            pltpu.SemaphoreType.DMA,
            pltpu.SemaphoreType.DMA,
            pltpu.SemaphoreType.DMA,
        ],
    )


def _proj_body(f0_ref, f1_ref, f2_ref, w0_ref, b0_ref, w1_ref, b1_ref,
               w2_ref, b2_ref, hist_ref, hh_ref, ns_ref, nd_ref):
    h0 = jnp.dot(f0_ref[...], w0_ref[...], preferred_element_type=jnp.float32) + b0_ref[...]
    h1 = jnp.dot(f1_ref[...], w1_ref[...], preferred_element_type=jnp.float32) + b1_ref[...]
    h2 = jnp.dot(f2_ref[...], w2_ref[...], preferred_element_type=jnp.float32) + b2_ref[...]
    h = jnp.concatenate([h0, h1, h2], axis=0)
    out_deg = hist_ref[0, :N_NODES, 0]
    in_deg = hist_ref[1, :N_NODES, 0]
    ns = lax.rsqrt(jnp.maximum(out_deg, 1.0))
    nd = lax.rsqrt(jnp.maximum(in_deg, 1.0))
    hh_ref[...] = h * ns[:, None]
    ns_ref[...] = ns[:, None]
    nd_ref[...] = nd[:, None]


def _scale_body(p_ref, nd_ref, ns_ref, out_ref):
    a = p_ref[0, :N_NODES, :] + p_ref[1, :N_NODES, :]
    out_ref[...] = jnp.maximum(a * nd_ref[...], 0.0) * ns_ref[...]


def _final_body(q_ref, nd_ref, w2_ref, b2_ref, out_ref):
    a = q_ref[0, :N_NODES, :] + q_ref[1, :N_NODES, :]
    a = a * nd_ref[...]
    z = jnp.dot(a, w2_ref[...], preferred_element_type=jnp.float32) + b2_ref[...]
    out_ref[...] = jnp.maximum(z, 0.0)


def kernel(feat0, feat1, feat2, edge_index, Wf0, bf0, Wf1, bf1, Wf2, bf2,
           W2, b2, semantic_weight):
    src = edge_index[0].astype(jnp.int32)
    dst = edge_index[1].astype(jnp.int32)
    n_pad_edges = N_EDGES_P - N_EDGES
    pad_iota = jnp.arange(n_pad_edges, dtype=jnp.int32)
    junk = N_NODES + pad_iota % (N_PAD - N_NODES)
    src_mp = jnp.concatenate([src, pad_iota % N_NODES])
    dst_mp = jnp.concatenate([dst, junk])
    srcr = src_mp.reshape(NW, NPAIR, 2, CH)
    dstr = dst_mp.reshape(NW, NPAIR, 2, CH)
    sd = jnp.stack([srcr, dstr], axis=3)
    ids = jnp.stack([jnp.concatenate([src, junk]), jnp.concatenate([dst, junk])])

    ones_deg = jnp.ones((DCH, DEG_W), jnp.float32)
    zeros_deg = jnp.zeros((DCH, DEG_W), jnp.float32)
    zeros_mp = jnp.zeros((CH, HID), jnp.float32)

    hist = _deg_kernel()(ids, ones_deg, zeros_deg)

    hh0, ns, nd = pl.pallas_call(
        _proj_body,
        out_shape=[
            jax.ShapeDtypeStruct((N_NODES, HID), jnp.float32),
            jax.ShapeDtypeStruct((N_NODES, 1), jnp.float32),
            jax.ShapeDtypeStruct((N_NODES, 1), jnp.float32),
        ],
    )(feat0, feat1, feat2, Wf0, bf0, Wf1, bf1, Wf2, bf2, hist)

    p0 = _msgpass_kernel()(hh0, sd, zeros_mp)

    hh1 = pl.pallas_call(
        _scale_body,
        out_shape=jax.ShapeDtypeStruct((N_NODES, HID), jnp.float32),
    )(p0, nd, ns)

    p1 = _msgpass_kernel()(hh1, sd, zeros_mp)

    out = pl.pallas_call(
        _final_body,
        out_shape=jax.ShapeDtypeStruct((N_NODES, HID), jnp.float32),
    )(p1, nd, W2, b2)

    return (out, semantic_weight)


# speedup vs baseline: 4.1700x; 1.3356x over previous
import functools

import jax
import jax.numpy as jnp
from jax import lax
from jax.experimental import pallas as pl
from jax.experimental.pallas import tpu as pltpu
from jax.experimental.pallas import tpu_sc as plsc

N_NODES = 10000
N_EDGES = 320000
HID = 128

NC = 2
NS = 16
NW = NC * NS

N_PAD = 10112
ROWS_PER_TILE = N_PAD // NS

CH = 128
E_TILE = 10240
NCH = E_TILE // CH
NPAIR = NCH // 2
N_EDGES_P = NW * E_TILE

D_TILE = N_EDGES_P // NS


@functools.cache
def _mesh():
    return plsc.VectorSubcoreMesh(
        core_axis_name="c", subcore_axis_name="s", num_cores=NC, num_subcores=NS
    )


_WB_CHUNKS = [(0, 128), (128, 128), (256, 128), (384, 128), (512, 120)]


def _deg_body(ids_hbm, out_hbm, idx_v, hist_v):
    c = lax.axis_index("c")
    s = lax.axis_index("s")

    def zero(i, carry):
        hist_v[pl.ds(i * 16, 16)] = jnp.zeros((16,), jnp.float32)
        return carry

    lax.fori_loop(0, N_PAD // 16, zero, 0)
    pltpu.sync_copy(ids_hbm.at[c, s], idx_v)
    ones = jnp.ones((16,), jnp.float32)

    def acc(i, carry):
        vec = idx_v[pl.ds(i * 16, 16)]
        plsc.addupdate_scatter(hist_v, [vec], ones)
        return carry

    lax.fori_loop(0, D_TILE // 16, acc, 0)
    pltpu.sync_copy(hist_v, out_hbm.at[c, s])


def _msgpass_body(h_hbm, sd_hbm, zeros_hbm, out_hbm,
                  sd_v, msg_v, agg_sh, sem_i, sem_g0, sem_g1, sem_s0, sem_s1):
    c = lax.axis_index("c")
    s = lax.axis_index("s")
    wid = c * NS + s
    row0 = s * ROWS_PER_TILE
    pltpu.sync_copy(zeros_hbm, msg_v.at[0])
    for off, sz in _WB_CHUNKS:
        pltpu.sync_copy(msg_v.at[0, pl.ds(0, sz)], agg_sh.at[pl.ds(row0 + off, sz)])
    plsc.subcore_barrier()

    def gstart(q, j, b, sem):
        pltpu.async_copy(h_hbm.at[sd_v.at[q, j, 0]], msg_v.at[b], sem)

    def gwait(sem):
        pltpu.make_async_copy(h_hbm.at[sd_v.at[0, 0, 0]], msg_v.at[0], sem).wait()

    def sstart(q, j, b, sem):
        pltpu.async_copy(msg_v.at[b], agg_sh.at[sd_v.at[q, j, 1]], sem, add=True)

    def swait(sem):
        pltpu.make_async_copy(msg_v.at[0], agg_sh.at[sd_v.at[0, 0, 1]], sem).wait()

    def istart(p, slot):
        pltpu.async_copy(sd_hbm.at[wid, p], sd_v.at[slot], sem_i)

    def iwait():
        pltpu.make_async_copy(sd_hbm.at[0, 0], sd_v.at[0], sem_i).wait()

    pltpu.sync_copy(sd_hbm.at[wid, 0], sd_v.at[0])
    gstart(0, 0, 0, sem_g0)
    istart(1, 1)
    gstart(0, 1, 1, sem_g1)
    gwait(sem_g0)
    sstart(0, 0, 0, sem_s0)
    iwait()
    swait(sem_s0)
    gstart(1, 0, 0, sem_g0)
    gwait(sem_g1)
    sstart(0, 1, 1, sem_s1)

    def body(p, carry):
        q = p % 2
        swait(sem_s1)
        istart(p + 1, 1 - q)
        gstart(q, 1, 1, sem_g1)
        gwait(sem_g0)
        sstart(q, 0, 0, sem_s0)
        iwait()
        swait(sem_s0)
        gstart(1 - q, 0, 0, sem_g0)
        gwait(sem_g1)
        sstart(q, 1, 1, sem_s1)
        return carry

    lax.fori_loop(1, NPAIR - 1, body, 0)

    swait(sem_s1)
    gstart(1, 1, 1, sem_g1)
    gwait(sem_g0)
    sstart(1, 0, 0, sem_s0)
    gwait(sem_g1)
    sstart(1, 1, 1, sem_s1)
    swait(sem_s0)
    swait(sem_s1)

    plsc.subcore_barrier()
    for off, sz in _WB_CHUNKS:
        pltpu.sync_copy(agg_sh.at[pl.ds(row0 + off, sz)], msg_v.at[0, pl.ds(0, sz)])
        pltpu.sync_copy(msg_v.at[0, pl.ds(0, sz)], out_hbm.at[c, pl.ds(row0 + off, sz)])


@functools.cache
def _deg_kernel():
    return pl.kernel(
        _deg_body,
        out_type=jax.ShapeDtypeStruct((NC, NS, N_PAD), jnp.float32),
        mesh=_mesh(),
        compiler_params=pltpu.CompilerParams(needs_layout_passes=False),
        scratch_types=[
            pltpu.VMEM((D_TILE,), jnp.int32),
            pltpu.VMEM((N_PAD,), jnp.float32),
        ],
    )


@functools.cache
def _msgpass_kernel():
    return pl.kernel(
        _msgpass_body,
        out_type=jax.ShapeDtypeStruct((NC, N_PAD, HID), jnp.float32),
        mesh=_mesh(),
        scratch_types=[
            pltpu.VMEM((2, 2, 2, CH), jnp.int32),
            pltpu.VMEM((2, CH, HID), jnp.float32),
            pltpu.VMEM_SHARED((N_PAD, HID), jnp.float32),
            pltpu.SemaphoreType.DMA,
            pltpu.SemaphoreType.DMA,
            pltpu.SemaphoreType.DMA,
            pltpu.SemaphoreType.DMA,
            pltpu.SemaphoreType.DMA,
        ],
    )


def _proj_body(f0_ref, f1_ref, f2_ref, w0_ref, b0_ref, w1_ref, b1_ref,
               w2_ref, b2_ref, hist_ref, hh_ref, ns_ref, nd_ref):
    h0 = jnp.dot(f0_ref[...], w0_ref[...], preferred_element_type=jnp.float32) + b0_ref[...]
    h1 = jnp.dot(f1_ref[...], w1_ref[...], preferred_element_type=jnp.float32) + b1_ref[...]
    h2 = jnp.dot(f2_ref[...], w2_ref[...], preferred_element_type=jnp.float32) + b2_ref[...]
    h = jnp.concatenate([h0, h1, h2], axis=0)
    out_deg = jnp.sum(hist_ref[0], axis=0)[:N_NODES]
    in_deg = jnp.sum(hist_ref[1], axis=0)[:N_NODES]
    ns = lax.rsqrt(jnp.maximum(out_deg, 1.0))
    nd = lax.rsqrt(jnp.maximum(in_deg, 1.0))
    hh_ref[...] = h * ns[:, None]
    ns_ref[...] = ns[:, None]
    nd_ref[...] = nd[:, None]


def _scale_body(p_ref, nd_ref, ns_ref, out_ref):
    a = p_ref[0, :N_NODES, :] + p_ref[1, :N_NODES, :]
    out_ref[...] = jnp.maximum(a * nd_ref[...], 0.0) * ns_ref[...]


def _final_body(q_ref, nd_ref, w2_ref, b2_ref, out_ref):
    a = q_ref[0, :N_NODES, :] + q_ref[1, :N_NODES, :]
    a = a * nd_ref[...]
    z = jnp.dot(a, w2_ref[...], preferred_element_type=jnp.float32) + b2_ref[...]
    out_ref[...] = jnp.maximum(z, 0.0)


def kernel(feat0, feat1, feat2, edge_index, Wf0, bf0, Wf1, bf1, Wf2, bf2,
           W2, b2, semantic_weight):
    src = edge_index[0].astype(jnp.int32)
    dst = edge_index[1].astype(jnp.int32)
    n_pad_edges = N_EDGES_P - N_EDGES
    pad_iota = jnp.arange(n_pad_edges, dtype=jnp.int32)
    junk = N_NODES + pad_iota % (N_PAD - N_NODES)
    src_mp = jnp.concatenate([src, pad_iota % N_NODES])
    dst_mp = jnp.concatenate([dst, junk])
    srcr = src_mp.reshape(NW, NPAIR, 2, CH)
    dstr = dst_mp.reshape(NW, NPAIR, 2, CH)
    sd = jnp.stack([srcr, dstr], axis=3)
    ids = jnp.stack([
        jnp.concatenate([src, junk]).reshape(NS, D_TILE),
        jnp.concatenate([dst, junk]).reshape(NS, D_TILE),
    ])

    zeros_mp = jnp.zeros((CH, HID), jnp.float32)

    hist = _deg_kernel()(ids)

    hh0, ns, nd = pl.pallas_call(
        _proj_body,
        out_shape=[
            jax.ShapeDtypeStruct((N_NODES, HID), jnp.float32),
            jax.ShapeDtypeStruct((N_NODES, 1), jnp.float32),
            jax.ShapeDtypeStruct((N_NODES, 1), jnp.float32),
        ],
    )(feat0, feat1, feat2, Wf0, bf0, Wf1, bf1, Wf2, bf2, hist)

    p0 = _msgpass_kernel()(hh0, sd, zeros_mp)

    hh1 = pl.pallas_call(
        _scale_body,
        out_shape=jax.ShapeDtypeStruct((N_NODES, HID), jnp.float32),
    )(p0, nd, ns)

    p1 = _msgpass_kernel()(hh1, sd, zeros_mp)

    out = pl.pallas_call(
        _final_body,
        out_shape=jax.ShapeDtypeStruct((N_NODES, HID), jnp.float32),
    )(p1, nd, W2, b2)

    return (out, semantic_weight)


# speedup vs baseline: 4.2200x; 1.0120x over previous
import functools

import jax
import jax.numpy as jnp
from jax import lax
from jax.experimental import pallas as pl
from jax.experimental.pallas import tpu as pltpu
from jax.experimental.pallas import tpu_sc as plsc

N_NODES = 10000
N_EDGES = 320000
HID = 128

NC = 2
NS = 16
NW = NC * NS

N_PAD = 10112
ROWS_PER_TILE = N_PAD // NS

CH = 128
E_TILE = 10240
NCH = E_TILE // CH
NPAIR = NCH // 2
N_EDGES_P = NW * E_TILE

D_TILE = N_EDGES_P // NS


@functools.cache
def _mesh():
    return plsc.VectorSubcoreMesh(
        core_axis_name="c", subcore_axis_name="s", num_cores=NC, num_subcores=NS
    )


_WB_CHUNKS = [(0, 128), (128, 128), (256, 128), (384, 128), (512, 120)]


def _deg_body(ids_hbm, out_hbm, idx_v, hist_v):
    c = lax.axis_index("c")
    s = lax.axis_index("s")

    def zero(i, carry):
        hist_v[pl.ds(i * 16, 16)] = jnp.zeros((16,), jnp.float32)
        return carry

    lax.fori_loop(0, N_PAD // 16, zero, 0)
    pltpu.sync_copy(ids_hbm.at[c, s], idx_v)
    ones = jnp.ones((16,), jnp.float32)

    def acc(i, carry):
        vec = idx_v[pl.ds(i * 16, 16)]
        plsc.addupdate_scatter(hist_v, [vec], ones)
        return carry

    lax.fori_loop(0, D_TILE // 16, acc, 0)
    pltpu.sync_copy(hist_v, out_hbm.at[c, s])


def _msgpass_body(h_hbm, sd_hbm, zeros_hbm, out_hbm,
                  sd_v, msg_v, agg_sh, sem_i, sem_g0, sem_g1, sem_s0, sem_s1):
    c = lax.axis_index("c")
    s = lax.axis_index("s")
    wid = c * NS + s
    row0 = s * ROWS_PER_TILE
    pltpu.sync_copy(zeros_hbm, msg_v.at[0])
    for off, sz in _WB_CHUNKS:
        pltpu.async_copy(msg_v.at[0, pl.ds(0, sz)], agg_sh.at[pl.ds(row0 + off, sz)], sem_i)
    for off, sz in _WB_CHUNKS:
        pltpu.make_async_copy(
            msg_v.at[0, pl.ds(0, sz)], agg_sh.at[pl.ds(row0 + off, sz)], sem_i
        ).wait()
    plsc.subcore_barrier()

    def gstart(q, j, b, sem):
        pltpu.async_copy(h_hbm.at[sd_v.at[q, j, 0]], msg_v.at[b], sem)

    def gwait(sem):
        pltpu.make_async_copy(h_hbm.at[sd_v.at[0, 0, 0]], msg_v.at[0], sem).wait()

    def sstart(q, j, b, sem):
        pltpu.async_copy(msg_v.at[b], agg_sh.at[sd_v.at[q, j, 1]], sem, add=True)

    def swait(sem):
        pltpu.make_async_copy(msg_v.at[0], agg_sh.at[sd_v.at[0, 0, 1]], sem).wait()

    def istart(p, slot):
        pltpu.async_copy(sd_hbm.at[wid, p], sd_v.at[slot], sem_i)

    def iwait():
        pltpu.make_async_copy(sd_hbm.at[0, 0], sd_v.at[0], sem_i).wait()

    pltpu.sync_copy(sd_hbm.at[wid, 0], sd_v.at[0])
    gstart(0, 0, 0, sem_g0)
    istart(1, 1)
    gstart(0, 1, 1, sem_g1)
    gwait(sem_g0)
    sstart(0, 0, 0, sem_s0)
    iwait()
    swait(sem_s0)
    gstart(1, 0, 0, sem_g0)
    gwait(sem_g1)
    sstart(0, 1, 1, sem_s1)

    def body(p, carry):
        q = p % 2
        swait(sem_s1)
        istart(p + 1, 1 - q)
        gstart(q, 1, 1, sem_g1)
        gwait(sem_g0)
        sstart(q, 0, 0, sem_s0)
        iwait()
        swait(sem_s0)
        gstart(1 - q, 0, 0, sem_g0)
        gwait(sem_g1)
        sstart(q, 1, 1, sem_s1)
        return carry

    lax.fori_loop(1, NPAIR - 1, body, 0)

    swait(sem_s1)
    gstart(1, 1, 1, sem_g1)
    gwait(sem_g0)
    sstart(1, 0, 0, sem_s0)
    gwait(sem_g1)
    sstart(1, 1, 1, sem_s1)
    swait(sem_s0)
    swait(sem_s1)

    plsc.subcore_barrier()

    def rstart(i):
        off, sz = _WB_CHUNKS[i]
        pltpu.async_copy(
            agg_sh.at[pl.ds(row0 + off, sz)], msg_v.at[i % 2, pl.ds(0, sz)], sem_g0)

    def rwait(i):
        off, sz = _WB_CHUNKS[i]
        pltpu.make_async_copy(
            agg_sh.at[pl.ds(row0 + off, sz)], msg_v.at[i % 2, pl.ds(0, sz)], sem_g0
        ).wait()

    def wstart(i):
        off, sz = _WB_CHUNKS[i]
        pltpu.async_copy(
            msg_v.at[i % 2, pl.ds(0, sz)], out_hbm.at[c, pl.ds(row0 + off, sz)], sem_g1)

    def wwait(i):
        off, sz = _WB_CHUNKS[i]
        pltpu.make_async_copy(
            msg_v.at[i % 2, pl.ds(0, sz)], out_hbm.at[c, pl.ds(row0 + off, sz)], sem_g1
        ).wait()

    n_wb = len(_WB_CHUNKS)
    rstart(0)
    for i in range(n_wb):
        rwait(i)
        wstart(i)
        if i + 1 < n_wb:
            if i >= 1:
                wwait(i - 1)
            rstart(i + 1)
    wwait(n_wb - 2)
    wwait(n_wb - 1)


@functools.cache
def _deg_kernel():
    return pl.kernel(
        _deg_body,
        out_type=jax.ShapeDtypeStruct((NC, NS, N_PAD), jnp.float32),
        mesh=_mesh(),
        compiler_params=pltpu.CompilerParams(needs_layout_passes=False),
        scratch_types=[
            pltpu.VMEM((D_TILE,), jnp.int32),
            pltpu.VMEM((N_PAD,), jnp.float32),
        ],
    )


@functools.cache
def _msgpass_kernel():
    return pl.kernel(
        _msgpass_body,
        out_type=jax.ShapeDtypeStruct((NC, N_PAD, HID), jnp.float32),
        mesh=_mesh(),
        scratch_types=[
            pltpu.VMEM((2, 2, 2, CH), jnp.int32),
            pltpu.VMEM((2, CH, HID), jnp.float32),
            pltpu.VMEM_SHARED((N_PAD, HID), jnp.float32),
            pltpu.SemaphoreType.DMA,
            pltpu.SemaphoreType.DMA,
            pltpu.SemaphoreType.DMA,
            pltpu.SemaphoreType.DMA,
            pltpu.SemaphoreType.DMA,
        ],
    )


def _proj_body(f0_ref, f1_ref, f2_ref, w0_ref, b0_ref, w1_ref, b1_ref,
               w2_ref, b2_ref, hist_ref, hh_ref, ns_ref, nd_ref):
    h0 = jnp.dot(f0_ref[...], w0_ref[...], preferred_element_type=jnp.float32) + b0_ref[...]
    h1 = jnp.dot(f1_ref[...], w1_ref[...], preferred_element_type=jnp.float32) + b1_ref[...]
    h2 = jnp.dot(f2_ref[...], w2_ref[...], preferred_element_type=jnp.float32) + b2_ref[...]
    h = jnp.concatenate([h0, h1, h2], axis=0)
    out_deg = jnp.sum(hist_ref[0], axis=0)[:N_NODES]
    in_deg = jnp.sum(hist_ref[1], axis=0)[:N_NODES]
    ns = lax.rsqrt(jnp.maximum(out_deg, 1.0))
    nd = lax.rsqrt(jnp.maximum(in_deg, 1.0))
    hh_ref[...] = h * ns[:, None]
    ns_ref[...] = ns[:, None]
    nd_ref[...] = nd[:, None]


def _scale_body(p_ref, nd_ref, ns_ref, out_ref):
    a = p_ref[0, :N_NODES, :] + p_ref[1, :N_NODES, :]
    out_ref[...] = jnp.maximum(a * nd_ref[...], 0.0) * ns_ref[...]


def _final_body(q_ref, nd_ref, w2_ref, b2_ref, out_ref):
    a = q_ref[0, :N_NODES, :] + q_ref[1, :N_NODES, :]
    a = a * nd_ref[...]
    z = jnp.dot(a, w2_ref[...], preferred_element_type=jnp.float32) + b2_ref[...]
    out_ref[...] = jnp.maximum(z, 0.0)


def kernel(feat0, feat1, feat2, edge_index, Wf0, bf0, Wf1, bf1, Wf2, bf2,
           W2, b2, semantic_weight):
    src = edge_index[0].astype(jnp.int32)
    dst = edge_index[1].astype(jnp.int32)
    n_pad_edges = N_EDGES_P - N_EDGES
    pad_iota = jnp.arange(n_pad_edges, dtype=jnp.int32)
    junk = N_NODES + pad_iota % (N_PAD - N_NODES)
    src_mp = jnp.concatenate([src, pad_iota % N_NODES])
    dst_mp = jnp.concatenate([dst, junk])
    srcr = src_mp.reshape(NW, NPAIR, 2, CH)
    dstr = dst_mp.reshape(NW, NPAIR, 2, CH)
    sd = jnp.stack([srcr, dstr], axis=3)
    ids = jnp.stack([
        jnp.concatenate([src, junk]).reshape(NS, D_TILE),
        jnp.concatenate([dst, junk]).reshape(NS, D_TILE),
    ])

    zeros_mp = jnp.zeros((CH, HID), jnp.float32)

    hist = _deg_kernel()(ids)

    hh0, ns, nd = pl.pallas_call(
        _proj_body,
        out_shape=[
            jax.ShapeDtypeStruct((N_NODES, HID), jnp.float32),
            jax.ShapeDtypeStruct((N_NODES, 1), jnp.float32),
            jax.ShapeDtypeStruct((N_NODES, 1), jnp.float32),
        ],
    )(feat0, feat1, feat2, Wf0, bf0, Wf1, bf1, Wf2, bf2, hist)

    p0 = _msgpass_kernel()(hh0, sd, zeros_mp)

    hh1 = pl.pallas_call(
        _scale_body,
        out_shape=jax.ShapeDtypeStruct((N_NODES, HID), jnp.float32),
    )(p0, nd, ns)

    p1 = _msgpass_kernel()(hh1, sd, zeros_mp)

    out = pl.pallas_call(
        _final_body,
        out_shape=jax.ShapeDtypeStruct((N_NODES, HID), jnp.float32),
    )(p1, nd, W2, b2)

    return (out, semantic_weight)


# speedup vs baseline: 4.2215x; 1.0004x over previous
import functools

import jax
import jax.numpy as jnp
from jax import lax
from jax.experimental import pallas as pl
from jax.experimental.pallas import tpu as pltpu
from jax.experimental.pallas import tpu_sc as plsc

N_NODES = 10000
N_EDGES = 320000
HID = 128

NC = 2
NS = 16
NW = NC * NS

N_PAD = 10112
ROWS_PER_TILE = N_PAD // NS

CH = 128
E_TILE = 10240
NCH = E_TILE // CH
NPAIR = NCH // 2
N_EDGES_P = NW * E_TILE

D_TILE = N_EDGES_P // NS


@functools.cache
def _mesh():
    return plsc.VectorSubcoreMesh(
        core_axis_name="c", subcore_axis_name="s", num_cores=NC, num_subcores=NS
    )


_WB_CHUNKS = [(0, 128), (128, 128), (256, 128), (384, 128), (512, 120)]


def _deg_body(ids_hbm, out_hbm, idx_v, hist_v):
    c = lax.axis_index("c")
    s = lax.axis_index("s")

    def zero(i, carry):
        hist_v[pl.ds(i * 16, 16)] = jnp.zeros((16,), jnp.float32)
        return carry

    lax.fori_loop(0, N_PAD // 16, zero, 0)
    pltpu.sync_copy(ids_hbm.at[c, s], idx_v)
    ones = jnp.ones((16,), jnp.float32)

    def acc(i, carry):
        vec = idx_v[pl.ds(i * 16, 16)]
        plsc.addupdate_scatter(hist_v, [vec], ones)
        return carry

    lax.fori_loop(0, D_TILE // 16, acc, 0)
    pltpu.sync_copy(hist_v, out_hbm.at[c, s])


def _msgpass_body(h_hbm, sd_hbm, zeros_hbm, out_hbm,
                  sd_v, msg_v, agg_sh, sem_i, sem_g0, sem_g1, sem_s0, sem_s1):
    c = lax.axis_index("c")
    s = lax.axis_index("s")
    wid = c * NS + s
    row0 = s * ROWS_PER_TILE
    pltpu.sync_copy(zeros_hbm, msg_v.at[0])
    for off, sz in _WB_CHUNKS:
        pltpu.async_copy(msg_v.at[0, pl.ds(0, sz)], agg_sh.at[pl.ds(row0 + off, sz)], sem_i)
    for off, sz in _WB_CHUNKS:
        pltpu.make_async_copy(
            msg_v.at[0, pl.ds(0, sz)], agg_sh.at[pl.ds(row0 + off, sz)], sem_i
        ).wait()
    plsc.subcore_barrier()

    def gstart(q, j, b, sem):
        pltpu.async_copy(h_hbm.at[sd_v.at[q, j, 0]], msg_v.at[b], sem)

    def gwait(sem):
        pltpu.make_async_copy(h_hbm.at[sd_v.at[0, 0, 0]], msg_v.at[0], sem).wait()

    def sstart(q, j, b, sem):
        pltpu.async_copy(msg_v.at[b], agg_sh.at[sd_v.at[q, j, 1]], sem, add=True)

    def swait(sem):
        pltpu.make_async_copy(msg_v.at[0], agg_sh.at[sd_v.at[0, 0, 1]], sem).wait()

    def istart(p, slot):
        pltpu.async_copy(sd_hbm.at[wid, p], sd_v.at[slot], sem_i)

    def iwait():
        pltpu.make_async_copy(sd_hbm.at[0, 0], sd_v.at[0], sem_i).wait()

    pltpu.sync_copy(sd_hbm.at[wid, 0], sd_v.at[0])
    gstart(0, 0, 0, sem_g0)
    istart(1, 1)
    gstart(0, 1, 1, sem_g1)
    gwait(sem_g0)
    sstart(0, 0, 0, sem_s0)
    iwait()
    swait(sem_s0)
    gstart(1, 0, 0, sem_g0)
    gwait(sem_g1)
    sstart(0, 1, 1, sem_s1)

    def body(p, carry):
        q = p % 2
        swait(sem_s1)
        istart(p + 1, 1 - q)
        gstart(q, 1, 1, sem_g1)
        gwait(sem_g0)
        sstart(q, 0, 0, sem_s0)
        iwait()
        swait(sem_s0)
        gstart(1 - q, 0, 0, sem_g0)
        gwait(sem_g1)
        sstart(q, 1, 1, sem_s1)
        return carry

    lax.fori_loop(1, NPAIR - 1, body, 0)

    swait(sem_s1)
    gstart(1, 1, 1, sem_g1)
    gwait(sem_g0)
    sstart(1, 0, 0, sem_s0)
    gwait(sem_g1)
    sstart(1, 1, 1, sem_s1)
    swait(sem_s0)
    swait(sem_s1)

    plsc.subcore_barrier()

    def rstart(i):
        off, sz = _WB_CHUNKS[i]
        pltpu.async_copy(
            agg_sh.at[pl.ds(row0 + off, sz)], msg_v.at[i % 2, pl.ds(0, sz)], sem_g0)

    def rwait(i):
        off, sz = _WB_CHUNKS[i]
        pltpu.make_async_copy(
            agg_sh.at[pl.ds(row0 + off, sz)], msg_v.at[i % 2, pl.ds(0, sz)], sem_g0
        ).wait()

    def wstart(i):
        off, sz = _WB_CHUNKS[i]
        pltpu.async_copy(
            msg_v.at[i % 2, pl.ds(0, sz)], out_hbm.at[c, pl.ds(row0 + off, sz)], sem_g1)

    def wwait(i):
        off, sz = _WB_CHUNKS[i]
        pltpu.make_async_copy(
            msg_v.at[i % 2, pl.ds(0, sz)], out_hbm.at[c, pl.ds(row0 + off, sz)], sem_g1
        ).wait()

    n_wb = len(_WB_CHUNKS)
    rstart(0)
    for i in range(n_wb):
        rwait(i)
        wstart(i)
        if i + 1 < n_wb:
            if i >= 1:
                wwait(i - 1)
            rstart(i + 1)
    wwait(n_wb - 2)
    wwait(n_wb - 1)


@functools.cache
def _deg_kernel():
    return pl.kernel(
        _deg_body,
        out_type=jax.ShapeDtypeStruct((NC, NS, N_PAD), jnp.float32),
        mesh=_mesh(),
        compiler_params=pltpu.CompilerParams(needs_layout_passes=False),
        scratch_types=[
            pltpu.VMEM((D_TILE,), jnp.int32),
            pltpu.VMEM((N_PAD,), jnp.float32),
        ],
    )


@functools.cache
def _msgpass_kernel():
    return pl.kernel(
        _msgpass_body,
        out_type=jax.ShapeDtypeStruct((NC, N_PAD, HID), jnp.float32),
        mesh=_mesh(),
        scratch_types=[
            pltpu.VMEM((2, 2, 2, CH), jnp.int32),
            pltpu.VMEM((2, CH, HID), jnp.float32),
            pltpu.VMEM_SHARED((N_PAD, HID), jnp.float32),
            pltpu.SemaphoreType.DMA,
            pltpu.SemaphoreType.DMA,
            pltpu.SemaphoreType.DMA,
            pltpu.SemaphoreType.DMA,
            pltpu.SemaphoreType.DMA,
        ],
    )


def _proj_body(f0_ref, f1_ref, f2_ref, w0_ref, b0_ref, w1_ref, b1_ref,
               w2_ref, b2_ref, h_ref):
    h0 = jnp.dot(f0_ref[...], w0_ref[...], preferred_element_type=jnp.float32) + b0_ref[...]
    h1 = jnp.dot(f1_ref[...], w1_ref[...], preferred_element_type=jnp.float32) + b1_ref[...]
    h2 = jnp.dot(f2_ref[...], w2_ref[...], preferred_element_type=jnp.float32) + b2_ref[...]
    h_ref[...] = jnp.concatenate([h0, h1, h2], axis=0)


def _norm_body(h_ref, hist_ref, hh_ref, ns_ref, nd_ref):
    out_deg = jnp.sum(hist_ref[0], axis=0)[:N_NODES]
    in_deg = jnp.sum(hist_ref[1], axis=0)[:N_NODES]
    ns = lax.rsqrt(jnp.maximum(out_deg, 1.0))
    nd = lax.rsqrt(jnp.maximum(in_deg, 1.0))
    hh_ref[...] = h_ref[...] * ns[:, None]
    ns_ref[...] = ns[:, None]
    nd_ref[...] = nd[:, None]


def _scale_body(p_ref, nd_ref, ns_ref, out_ref):
    a = p_ref[0, :N_NODES, :] + p_ref[1, :N_NODES, :]
    out_ref[...] = jnp.maximum(a * nd_ref[...], 0.0) * ns_ref[...]


def _final_body(q_ref, nd_ref, w2_ref, b2_ref, out_ref):
    a = q_ref[0, :N_NODES, :] + q_ref[1, :N_NODES, :]
    a = a * nd_ref[...]
    z = jnp.dot(a, w2_ref[...], preferred_element_type=jnp.float32) + b2_ref[...]
    out_ref[...] = jnp.maximum(z, 0.0)


def kernel(feat0, feat1, feat2, edge_index, Wf0, bf0, Wf1, bf1, Wf2, bf2,
           W2, b2, semantic_weight):
    src = edge_index[0].astype(jnp.int32)
    dst = edge_index[1].astype(jnp.int32)
    n_pad_edges = N_EDGES_P - N_EDGES
    pad_iota = jnp.arange(n_pad_edges, dtype=jnp.int32)
    junk = N_NODES + pad_iota % (N_PAD - N_NODES)
    src_mp = jnp.concatenate([src, pad_iota % N_NODES])
    dst_mp = jnp.concatenate([dst, junk])
    srcr = src_mp.reshape(NW, NPAIR, 2, CH)
    dstr = dst_mp.reshape(NW, NPAIR, 2, CH)
    sd = jnp.stack([srcr, dstr], axis=3)
    ids = jnp.stack([
        jnp.concatenate([src, junk]).reshape(NS, D_TILE),
        jnp.concatenate([dst, junk]).reshape(NS, D_TILE),
    ])

    zeros_mp = jnp.zeros((CH, HID), jnp.float32)

    hist = _deg_kernel()(ids)

    h = pl.pallas_call(
        _proj_body,
        out_shape=jax.ShapeDtypeStruct((N_NODES, HID), jnp.float32),
    )(feat0, feat1, feat2, Wf0, bf0, Wf1, bf1, Wf2, bf2)

    hh0, ns, nd = pl.pallas_call(
        _norm_body,
        out_shape=[
            jax.ShapeDtypeStruct((N_NODES, HID), jnp.float32),
            jax.ShapeDtypeStruct((N_NODES, 1), jnp.float32),
            jax.ShapeDtypeStruct((N_NODES, 1), jnp.float32),
        ],
    )(h, hist)

    p0 = _msgpass_kernel()(hh0, sd, zeros_mp)

    hh1 = pl.pallas_call(
        _scale_body,
        out_shape=jax.ShapeDtypeStruct((N_NODES, HID), jnp.float32),
    )(p0, nd, ns)

    p1 = _msgpass_kernel()(hh1, sd, zeros_mp)

    out = pl.pallas_call(
        _final_body,
        out_shape=jax.ShapeDtypeStruct((N_NODES, HID), jnp.float32),
    )(p1, nd, W2, b2)

    return (out, semantic_weight)
